# baseline (speedup 1.0000x reference)
import jax
import jax.numpy as jnp
from jax.experimental import pallas as pl
from jax.experimental.pallas import tpu as pltpu

_B = 4
_S = 8192
_D = 2048
_N = 64
_STRIDE = 128
_ROWS = _B * _N


_CHUNKS = [(0, 0, 16), (0, 16, 48), (1, 0, _N), (2, 0, _N), (3, 0, _N)]


def _read(x_hbm, buf, rsem, c):
    b, o, n = _CHUNKS[c]
    return pltpu.make_async_copy(
        x_hbm.at[b, pl.ds(o, n), 0, :],
        buf.at[pl.ds(b * _N + o, n)],
        rsem.at[c],
    )


def _write(buf, out_hbm, wsem, c):
    b, o, n = _CHUNKS[c]
    return pltpu.make_async_copy(
        buf.at[pl.ds(b * _N + o, n)],
        out_hbm.at[pl.ds(b * _N + o, n)],
        wsem.at[c],
    )


def _gather_body(x_hbm, out_hbm, buf, rsem, wsem):
    for c in range(len(_CHUNKS)):
        _read(x_hbm, buf, rsem, c).start()
    for c in range(len(_CHUNKS)):
        _read(x_hbm, buf, rsem, c).wait()
        _write(buf, out_hbm, wsem, c).start()
    for c in range(len(_CHUNKS)):
        _write(buf, out_hbm, wsem, c).wait()


def kernel(x):
    x4 = x.reshape(_B, _N, _STRIDE, _D)
    out = pl.pallas_call(
        _gather_body,
        in_specs=[pl.BlockSpec(memory_space=pl.ANY)],
        out_specs=pl.BlockSpec(memory_space=pl.ANY),
        out_shape=jax.ShapeDtypeStruct((_ROWS, _D), jnp.float32),
        scratch_shapes=[
            pltpu.VMEM((_ROWS, _D), jnp.float32),
            pltpu.SemaphoreType.DMA((len(_CHUNKS),)),
            pltpu.SemaphoreType.DMA((len(_CHUNKS),)),
        ],
    )(x4)
    return out.reshape(_B, _N, _D)


# speedup vs baseline: 1.0022x; 1.0022x over previous
import jax
import jax.numpy as jnp
from jax.experimental import pallas as pl
from jax.experimental.pallas import tpu as pltpu

_B = 4
_S = 8192
_D = 2048
_N = 64
_STRIDE = 128
_ROWS = _B * _N


def _read(x_hbm, buf, rsem, b):
    return pltpu.make_async_copy(
        x_hbm.at[b, :, 0, :],
        buf.at[pl.ds(b * _N, _N)],
        rsem.at[b],
    )


def _write(buf, out_hbm, wsem, b):
    return pltpu.make_async_copy(
        buf.at[pl.ds(b * _N, _N)],
        out_hbm.at[pl.ds(b * _N, _N)],
        wsem.at[b],
    )


def _gather_body(x_hbm, out_hbm, buf, rsem, wsem):
    for b in range(_B):
        _read(x_hbm, buf, rsem, b).start()
    for b in range(_B):
        _read(x_hbm, buf, rsem, b).wait()
        _write(buf, out_hbm, wsem, b).start()
    for b in range(_B):
        _write(buf, out_hbm, wsem, b).wait()


def kernel(x):
    x4 = x.reshape(_B, _N, _STRIDE, _D)
    out = pl.pallas_call(
        _gather_body,
        in_specs=[pl.BlockSpec(memory_space=pl.ANY)],
        out_specs=pl.BlockSpec(memory_space=pl.ANY),
        out_shape=jax.ShapeDtypeStruct((_ROWS, _D), jnp.float32),
        scratch_shapes=[
            pltpu.VMEM((_ROWS, _D), jnp.float32),
            pltpu.SemaphoreType.DMA((_B,)),
            pltpu.SemaphoreType.DMA((_B,)),
        ],
    )(x4)
    return out.reshape(_B, _N, _D)
